# Initial kernel scaffold; baseline (speedup 1.0000x reference)
import jax
import jax.numpy as jnp
from jax.experimental import pallas as pl


def kernel(sentence, table):
    raise NotImplementedError("write your pallas kernel here")


# speedup vs baseline: 2.4868x; 2.4868x over previous
import functools

import jax
import jax.numpy as jnp
from jax import lax
from jax.experimental import pallas as pl
from jax.experimental.pallas import tpu as pltpu
from jax.experimental.pallas import tpu_sc as plsc

_INFO = plsc.get_sparse_core_info()
_NC, _NS = _INFO.num_cores, _INFO.num_subcores
_NW = _NC * _NS

_D = 16
_CHUNK = 2048


@functools.partial(jax.jit, static_argnames=("n_total",))
def _lookup(idx_flat, table, *, n_total):
    n_per_w = n_total // _NW
    n_chunks = n_per_w // _CHUNK
    assert n_per_w % _CHUNK == 0

    mesh = plsc.VectorSubcoreMesh(core_axis_name="c", subcore_axis_name="s")

    @functools.partial(
        pl.kernel,
        out_type=jax.ShapeDtypeStruct((n_total, _D), jnp.float32),
        mesh=mesh,
        scratch_types=[
            pltpu.VMEM((_CHUNK,), jnp.int32),
            pltpu.VMEM((_CHUNK, _D), jnp.float32),
            pltpu.SemaphoreType.DMA,
        ],
        compiler_params=pltpu.CompilerParams(use_tc_tiling_on_sc=False),
    )
    def body(idx_hbm, table_hbm, out_hbm, idx_v, rows_v, sem):
        wid = lax.axis_index("s") * _NC + lax.axis_index("c")
        base = wid * n_per_w

        def step(i, carry):
            off = base + i * _CHUNK
            pltpu.sync_copy(idx_hbm.at[pl.ds(off, _CHUNK)], idx_v)
            pltpu.async_copy(table_hbm.at[idx_v], rows_v, sem).wait()
            pltpu.sync_copy(rows_v, out_hbm.at[pl.ds(off, _CHUNK)])
            return carry

        lax.fori_loop(0, n_chunks, step, 0)

    return body(idx_flat, table)


def kernel(sentence, table):
    b, h = sentence.shape
    idx_flat = sentence.reshape(-1).astype(jnp.int32)
    out = _lookup(idx_flat, table, n_total=b * h)
    return out.reshape(b, h, table.shape[1])


# speedup vs baseline: 2.5441x; 1.0231x over previous
import functools

import jax
import jax.numpy as jnp
from jax import lax
from jax.experimental import pallas as pl
from jax.experimental.pallas import tpu as pltpu
from jax.experimental.pallas import tpu_sc as plsc

_INFO = plsc.get_sparse_core_info()
_NC, _NS = _INFO.num_cores, _INFO.num_subcores
_NW = _NC * _NS

_D = 16
_CHUNK = 3200


@functools.partial(jax.jit, static_argnames=("n_total",))
def _lookup(idx_flat, table, *, n_total):
    n_per_w = n_total // _NW
    n_chunks = n_per_w // _CHUNK
    assert n_per_w % _CHUNK == 0 and n_chunks % 2 == 0 and n_chunks >= 4

    mesh = plsc.VectorSubcoreMesh(core_axis_name="c", subcore_axis_name="s")

    @functools.partial(
        pl.kernel,
        out_type=jax.ShapeDtypeStruct((n_total, _D), jnp.float32),
        mesh=mesh,
        scratch_types=[
            pltpu.VMEM((2, _CHUNK), jnp.int32),
            pltpu.VMEM((2, _CHUNK, _D), jnp.float32),
            pltpu.SemaphoreType.DMA,
            pltpu.SemaphoreType.DMA,
            pltpu.SemaphoreType.DMA,
            pltpu.SemaphoreType.DMA,
            pltpu.SemaphoreType.DMA,
            pltpu.SemaphoreType.DMA,
        ],
        compiler_params=pltpu.CompilerParams(use_tc_tiling_on_sc=False),
    )
    def body(idx_hbm, table_hbm, out_hbm, idx_v, rows_v, si0, si1, sg0, sg1,
             so0, so1):
        wid = lax.axis_index("s") * _NC + lax.axis_index("c")
        base = wid * n_per_w
        si = (si0, si1)
        sg = (sg0, sg1)
        so = (so0, so1)

        def idx_start(c, b):
            pltpu.async_copy(idx_hbm.at[pl.ds(base + c * _CHUNK, _CHUNK)],
                             idx_v.at[b], si[b])

        def idx_wait(b):
            pltpu.make_async_copy(idx_hbm.at[pl.ds(base, _CHUNK)],
                                  idx_v.at[b], si[b]).wait()

        def g_start(b):
            pltpu.async_copy(table_hbm.at[idx_v.at[b]], rows_v.at[b], sg[b])

        def g_wait(b):
            pltpu.make_async_copy(table_hbm.at[idx_v.at[b]], rows_v.at[b],
                                  sg[b]).wait()

        def out_start(c, b):
            pltpu.async_copy(rows_v.at[b],
                             out_hbm.at[pl.ds(base + c * _CHUNK, _CHUNK)],
                             so[b])

        def out_wait(b):
            pltpu.make_async_copy(rows_v.at[b],
                                  out_hbm.at[pl.ds(base, _CHUNK)],
                                  so[b]).wait()

        idx_start(0, 0)
        idx_start(1, 1)
        idx_wait(0)
        g_start(0)
        g_wait(0)
        out_start(0, 0)
        idx_start(2, 0)
        idx_wait(1)
        g_start(1)
        g_wait(1)
        out_start(1, 1)
        idx_start(3, 1)
        idx_wait(0)
        out_wait(0)
        g_start(0)

        def pair(g, carry):
            c0 = 2 * g
            g_wait(0)
            out_start(c0, 0)
            idx_start(c0 + 2, 0)
            idx_wait(1)
            out_wait(1)
            g_start(1)
            g_wait(1)
            out_start(c0 + 1, 1)
            idx_start(c0 + 3, 1)
            idx_wait(0)
            out_wait(0)
            g_start(0)
            return carry

        lax.fori_loop(1, n_chunks // 2 - 1, pair, 0)

        g_wait(0)
        out_start(n_chunks - 2, 0)
        idx_wait(1)
        out_wait(1)
        g_start(1)
        g_wait(1)
        out_start(n_chunks - 1, 1)
        out_wait(0)
        out_wait(1)

    return body(idx_flat, table)


def kernel(sentence, table):
    b, h = sentence.shape
    idx_flat = sentence.reshape(-1).astype(jnp.int32)
    out = _lookup(idx_flat, table, n_total=b * h)
    return out.reshape(b, h, table.shape[1])


# speedup vs baseline: 4.9940x; 1.9630x over previous
import functools

import jax
import jax.numpy as jnp
from jax import lax
from jax.experimental import pallas as pl
from jax.experimental.pallas import tpu as pltpu
from jax.experimental.pallas import tpu_sc as plsc

_INFO = plsc.get_sparse_core_info()
_NC, _NS = _INFO.num_cores, _INFO.num_subcores
_NW = _NC * _NS

_D = 16
_C = 1024
_BATCH = 16384
_HIST = 200
_GRP = _BATCH // _C
_NCHUNK = _HIST * _GRP
_PER_W = _NCHUNK // _NW


def _build():
    mesh = plsc.VectorSubcoreMesh(core_axis_name="c", subcore_axis_name="s")
    n_rows = _HIST * 2 * (_BATCH // 128) * 8

    @functools.partial(
        pl.kernel,
        out_type=jax.ShapeDtypeStruct((n_rows, 128), jnp.float32),
        mesh=mesh,
        scratch_types=[
            pltpu.VMEM((_C,), jnp.int32),
            pltpu.VMEM((_C,), jnp.int32),
            pltpu.VMEM((_C, _D), jnp.float32),
            pltpu.VMEM((_C, _D), jnp.float32),
            pltpu.VMEM((2, 64, 128), jnp.float32),
            pltpu.VMEM((2, 64, 128), jnp.float32),
            pltpu.SemaphoreType.DMA,
            pltpu.SemaphoreType.DMA,
            pltpu.SemaphoreType.DMA,
            pltpu.SemaphoreType.DMA,
            pltpu.SemaphoreType.DMA,
            pltpu.SemaphoreType.DMA,
        ],
        compiler_params=pltpu.CompilerParams(use_tc_tiling_on_sc=False,
                                             needs_layout_passes=False),
    )
    def body(flat_hbm, table_hbm, out_hbm, idx0, idx1, rows0, rows1, tb0, tb1,
             si0, si1, sg0, sg1, so0, so1):
        wid = lax.axis_index("s") * _NC + lax.axis_index("c")
        c_base = wid * _PER_W
        idxs = (idx0, idx1)
        rows = (rows0, rows1)
        tbs = (tb0, tb1)
        si = (si0, si1)
        sg = (sg0, sg1)
        so = (so0, so1)
        iota = lax.iota(jnp.int32, 16)

        def idx_start(c, b):
            pltpu.async_copy(flat_hbm.at[pl.ds((c_base + c) * _C, _C)],
                             idxs[b], si[b])

        def idx_wait(b):
            pltpu.make_async_copy(flat_hbm.at[pl.ds(0, _C)], idxs[b],
                                  si[b]).wait()

        def g_start(b):
            pltpu.async_copy(table_hbm.at[idxs[b]], rows[b], sg[b])

        def g_wait(b):
            pltpu.make_async_copy(table_hbm.at[idxs[b]], rows[b],
                                  sg[b]).wait()

        def out_start(c, b):
            cg = c_base + c
            h = cg // _GRP
            btg = cg - h * _GRP
            for dt in range(2):
                row0 = h * 2048 + dt * 1024 + btg * 64
                pltpu.async_copy(tbs[b].at[dt], out_hbm.at[pl.ds(row0, 64)],
                                 so[b])

        def out_wait(b):
            for dt in range(2):
                pltpu.make_async_copy(tbs[b].at[dt],
                                      out_hbm.at[pl.ds(0, 64)], so[b]).wait()

        def transpose(b):
            rb = rows[b]
            tb = tbs[b]

            def tloop(bt, carry):
                t_base = bt * 128 + iota
                for dt in range(2):
                    for s in range(8):
                        d_idx = jnp.full((16,), dt * 8 + s, dtype=jnp.int32)
                        for l0 in range(8):
                            v = plsc.load_gather(rb, [t_base + l0 * 16, d_idx])
                            tb[dt, bt * 8 + s, pl.ds(l0 * 16, 16)] = v
                return carry

            lax.fori_loop(0, 8, tloop, 0)

        idx_start(0, 0)
        idx_start(1, 1)
        idx_wait(0)
        g_start(0)

        g_wait(0)
        idx_start(2, 0)
        idx_wait(1)
        g_start(1)
        transpose(0)
        out_start(0, 0)

        g_wait(1)
        idx_start(3, 1)
        idx_wait(0)
        g_start(0)
        transpose(1)
        out_start(1, 1)

        def pair(g, carry):
            c0 = 2 * g
            g_wait(0)
            idx_start(c0 + 2, 0)
            idx_wait(1)
            g_start(1)
            out_wait(0)
            transpose(0)
            out_start(c0, 0)

            g_wait(1)
            idx_start(c0 + 3, 1)
            idx_wait(0)
            g_start(0)
            out_wait(1)
            transpose(1)
            out_start(c0 + 1, 1)
            return carry

        lax.fori_loop(1, _PER_W // 2 - 1, pair, 0)

        g_wait(0)
        idx_wait(1)
        g_start(1)
        out_wait(0)
        transpose(0)
        out_start(_PER_W - 2, 0)

        g_wait(1)
        out_wait(1)
        transpose(1)
        out_start(_PER_W - 1, 1)

        out_wait(0)
        out_wait(1)

    return body


_LOOKUP = _build()


def kernel(sentence, table):
    b, h = sentence.shape
    d = table.shape[1]
    flat_t = sentence.T.reshape(-1).astype(jnp.int32)
    out2 = _LOOKUP(flat_t, table)
    out = out2.reshape(h, 2, b // 128, 8, 128).transpose(2, 4, 0, 1, 3)
    return out.reshape(b, h, d)
